# baseline (speedup 1.0000x reference)
import functools
import jax
import jax.numpy as jnp
from jax import lax
from jax.experimental import pallas as pl
from jax.experimental.pallas import tpu as pltpu
from jax.experimental.pallas import tpu_sc as plsc

_EPS = 1e-5
_NG = 64
_F = 1024
_N = 10000
_E = 160000

_NW = 32
_NT = 313
_NPAD = _NW * _NT
_ECH = 4000
_NCH = _E // _ECH
_G = 128
_FC = 128
_NC_F = _F // _FC
_HROW = 5 * _NC_F


def _mm_block(a_ref, b_ref, o_ref):
    o_ref[...] = jnp.dot(a_ref[...], b_ref[...],
                         preferred_element_type=jnp.float32)


def _matmul(a, b, bm, bn):
    m, k = a.shape
    _, n = b.shape
    return pl.pallas_call(
        _mm_block,
        grid=(m // bm, n // bn),
        in_specs=[
            pl.BlockSpec((bm, k), lambda i, j: (i, 0)),
            pl.BlockSpec((k, bn), lambda i, j: (0, j)),
        ],
        out_specs=pl.BlockSpec((bm, bn), lambda i, j: (i, j)),
        out_shape=jax.ShapeDtypeStruct((m, n), jnp.float32),
    )(a, b)


def _sc_body(hflat, dst_h, src_h, sigs_h,
             m1, s1, m2, s2,
             dbuf, sbuf, dcomp, scomp, acc_m, acc_s, gbuf, ibuf, sigv,
             sem):
    wid = lax.axis_index("s") * 2 + lax.axis_index("c")
    base = wid * _NT
    lanes = lax.iota(jnp.int32, 16)

    pltpu.sync_copy(sigs_h, sigv)

    def do_pass(p, _):
        v = p >> 3
        c = p & 7
        toff = 16 + 16 * v + c

        def init_row(r, _):
            for j in range(_NC_F):
                acc_m[r, 0, pl.ds(16 * j, 16)] = jnp.full((16,), -3.0e38,
                                                          jnp.float32)
                acc_s[r, 0, pl.ds(16 * j, 16)] = jnp.zeros((16,), jnp.float32)
            return 0
        lax.fori_loop(0, _NT, init_row, 0)

        sgv = [sigv[v, c, pl.ds(16 * j, 16)] for j in range(_NC_F)]

        def do_chunk(k, _):
            pltpu.sync_copy(dst_h.at[pl.ds(k * _ECH, _ECH)], dbuf)
            pltpu.sync_copy(src_h.at[pl.ds(k * _ECH, _ECH)], sbuf)

            def scan(i, cnt_vec):
                d = dbuf[pl.ds(16 * i, 16)]
                s = sbuf[pl.ds(16 * i, 16)]
                msk = (d >= base) & (d < base + _NT)
                m32 = msk.astype(jnp.int32)
                pos = cnt_vec + plsc.cumsum(m32) - 1
                plsc.store_scatter(dcomp, [pos], d - base, mask=msk)
                plsc.store_scatter(scomp, [pos], s, mask=msk)
                return cnt_vec + plsc.all_reduce_population_count(msk)
            cnt_vec = lax.fori_loop(0, _ECH // 16, scan,
                                    jnp.zeros((16,), jnp.int32))
            cnt = jnp.max(cnt_vec)

            def do_group(g, _):
                gl = jnp.minimum(_G, cnt - g * _G)
                for j in range(_G // 16):
                    s16 = scomp[pl.ds(g * _G + 16 * j, 16)]
                    valid = (g * _G + 16 * j + lanes) < cnt
                    idx = jnp.where(valid, s16 * _HROW + toff, 0)
                    ibuf[pl.ds(16 * j, 16)] = idx
                pltpu.async_copy(hflat.at[ibuf], gbuf, sem).wait()

                def do_edge(e, _):
                    dl = dcomp[pl.ds(g * _G + e, 16)][0]
                    for j in range(_NC_F):
                        b = gbuf[e, pl.ds(16 * j, 16)]
                        bs = b * sgv[j]
                        cur = acc_m[dl, 0, pl.ds(16 * j, 16)]
                        acc_m[dl, 0, pl.ds(16 * j, 16)] = jnp.maximum(cur, bs)
                        plsc.addupdate(acc_s.at[dl, 0, pl.ds(16 * j, 16)], b)
                    return 0
                lax.fori_loop(0, gl, do_edge, 0)
                return 0

            ng = (cnt + _G - 1) // _G
            lax.fori_loop(0, ng, do_group, 0)
            return 0

        lax.fori_loop(0, _NCH, do_chunk, 0)

        @pl.when(v == 0)
        def _():
            pltpu.sync_copy(acc_m.at[pl.ds(0, _NT)],
                            m1.at[pl.ds(base, _NT), pl.ds(c, 1)])
            pltpu.sync_copy(acc_s.at[pl.ds(0, _NT)],
                            s1.at[pl.ds(base, _NT), pl.ds(c, 1)])

        @pl.when(v == 1)
        def _():
            pltpu.sync_copy(acc_m.at[pl.ds(0, _NT)],
                            m2.at[pl.ds(base, _NT), pl.ds(c, 1)])
            pltpu.sync_copy(acc_s.at[pl.ds(0, _NT)],
                            s2.at[pl.ds(base, _NT), pl.ds(c, 1)])
        return 0

    lax.fori_loop(0, 2 * _NC_F, do_pass, 0)


_out_sd = jax.ShapeDtypeStruct((_NPAD, _NC_F, _FC), jnp.float32)

_sc_call = functools.partial(
    pl.kernel,
    mesh=plsc.VectorSubcoreMesh(core_axis_name="c", subcore_axis_name="s"),
    compiler_params=pltpu.CompilerParams(needs_layout_passes=False),
    out_type=[_out_sd, _out_sd, _out_sd, _out_sd],
    scratch_types=[
        pltpu.VMEM((_ECH,), jnp.int32),
        pltpu.VMEM((_ECH,), jnp.int32),
        pltpu.VMEM((_ECH + 16,), jnp.int32),
        pltpu.VMEM((_ECH + 16,), jnp.int32),
        pltpu.VMEM((_NT, 1, _FC), jnp.float32),
        pltpu.VMEM((_NT, 1, _FC), jnp.float32),
        pltpu.VMEM((_G, _FC), jnp.float32),
        pltpu.VMEM((_G,), jnp.int32),
        pltpu.VMEM((2, _NC_F, _FC), jnp.float32),
        pltpu.SemaphoreType.DMA,
    ],
)(_sc_body)


def kernel(x, edge_index, batch, W0, b0, g0, be0, W1, b1, g1, be1,
           W2, b2, g2, be2, Wr, br):
    src = edge_index[0]
    dst = edge_index[1]

    Wcat = jnp.concatenate([
        W0,
        W1[:_F] - W1[_F:], W1[_F:],
        W2[:_F] - W2[_F:], W2[_F:],
    ], axis=1)
    H = _matmul(x, Wcat, 400, 512)
    H0 = H[:, 0 * _F:1 * _F]
    A1 = H[:, 1 * _F:2 * _F]
    A2 = H[:, 3 * _F:4 * _F]

    hflat = H.reshape(_N * _HROW, _FC)
    sigs = jnp.stack([jnp.sign(g1).reshape(_NC_F, _FC),
                      jnp.sign(g2).reshape(_NC_F, _FC)])
    m1p, s1p, m2p, s2p = _sc_call(hflat, dst, src, sigs)
    M1 = m1p.reshape(_NPAD, _F)[:_N]
    S1 = s1p.reshape(_NPAD, _F)[:_N]
    M2 = m2p.reshape(_NPAD, _F)[:_N]
    S2 = s2p.reshape(_NPAD, _F)[:_N]

    cnt_dst = jax.ops.segment_sum(jnp.ones((_E,), jnp.float32), dst,
                                  num_segments=_N)
    cnt_src = jax.ops.segment_sum(jnp.ones((_E,), jnp.float32), src,
                                  num_segments=_N)

    P = (batch[None, :] == jnp.arange(_NG)[:, None]).astype(jnp.float32)
    Pn = P / jnp.maximum(P.sum(1, keepdims=True), 1.0)

    mu0 = jnp.mean(H0, axis=0) + b0
    var0 = jnp.mean((H0 + b0[None, :] - mu0[None, :]) ** 2, axis=0)
    s0 = g0 * jax.lax.rsqrt(var0 + _EPS)
    t0 = (b0 - mu0) * s0 + be0
    p0 = (Pn @ H0) * s0[None, :] + t0[None, :]

    def edgeconv2(A, B, S, M, b, g, be):
        sumA = cnt_dst @ A
        sumB = cnt_src @ B
        sumA2 = cnt_dst @ (A * A)
        sumB2 = cnt_src @ (B * B)
        cross = jnp.sum(A * S, axis=0)
        mu_nb = (sumA + sumB) / _E
        var = (sumA2 + sumB2 + 2.0 * cross) / _E - mu_nb ** 2
        s = g * jax.lax.rsqrt(var + _EPS)
        t = (b - mu_nb) * s + be
        mask = (cnt_dst > 0)[:, None]
        Mz = jnp.where(mask, M, 0.0)
        agg = A * s[None, :] + t[None, :] + jnp.abs(s)[None, :] * Mz
        agg = jnp.where(mask & jnp.isfinite(agg), agg, 0.0)
        return Pn @ agg

    B1 = H[:, 2 * _F:3 * _F]
    B2 = H[:, 4 * _F:5 * _F]
    p1 = edgeconv2(A1, B1, S1, M1, b1, g1, be1)
    p2 = edgeconv2(A2, B2, S2, M2, b2, g2, be2)
    acc = p0 + p1 + p2
    return acc @ Wr + br


# speedup vs baseline: 4.3624x; 4.3624x over previous
import functools
import jax
import jax.numpy as jnp
from jax import lax
from jax.experimental import pallas as pl
from jax.experimental.pallas import tpu as pltpu
from jax.experimental.pallas import tpu_sc as plsc

_EPS = 1e-5
_NG = 64
_F = 1024
_N = 10000
_E = 160000

_NW = 32
_NT = 313
_NPAD = _NW * _NT
_ECH = 4000
_NCH = _E // _ECH
_CAP = 6000
_G = 128
_FC = 128
_NC_F = _F // _FC
_HROW = 5 * _NC_F


def _mm_block(a_ref, b_ref, o_ref):
    o_ref[...] = jnp.dot(a_ref[...], b_ref[...],
                         preferred_element_type=jnp.float32)


def _matmul(a, b, bm, bn):
    m, k = a.shape
    _, n = b.shape
    return pl.pallas_call(
        _mm_block,
        grid=(m // bm, n // bn),
        in_specs=[
            pl.BlockSpec((bm, k), lambda i, j: (i, 0)),
            pl.BlockSpec((k, bn), lambda i, j: (0, j)),
        ],
        out_specs=pl.BlockSpec((bm, bn), lambda i, j: (i, j)),
        out_shape=jax.ShapeDtypeStruct((m, n), jnp.float32),
    )(a, b)


def _sc_body(hflat, dst_h, src_h, sigs_h,
             m1, s1, m2, s2,
             dbuf, sbuf, dcomp, icomp, acc_m, acc_s, gbuf, sigv,
             sem):
    wid = lax.axis_index("s") * 2 + lax.axis_index("c")
    base = wid * _NT

    pltpu.sync_copy(sigs_h, sigv)

    def initc(i, _):
        icomp[pl.ds(16 * i, 16)] = jnp.zeros((16,), jnp.int32)
        return 0
    lax.fori_loop(0, (_CAP + 128) // 16, initc, 0)

    def do_pass(p, _):
        v = p >> 3
        c = p & 7
        toff = 16 + 16 * v + c

        def init_row(r, _):
            for j in range(_NC_F):
                acc_m[r, 0, pl.ds(16 * j, 16)] = jnp.full((16,), -3.0e38,
                                                          jnp.float32)
                acc_s[r, 0, pl.ds(16 * j, 16)] = jnp.zeros((16,), jnp.float32)
            return 0
        lax.fori_loop(0, _NT, init_row, 0)

        sgv = [sigv[v, c, pl.ds(16 * j, 16)] for j in range(_NC_F)]

        def flush(cnt):
            ng = (cnt + _G - 1) // _G

            def do_group(g, _):
                pltpu.async_copy(hflat.at[icomp.at[pl.ds(g * _G, _G)]],
                                 gbuf, sem).wait()
                gl = jnp.minimum(_G, cnt - g * _G)

                def do_edge(e, _):
                    dl = dcomp[pl.ds(g * _G + e, 16)][0]
                    for j in range(_NC_F):
                        b = gbuf[e, pl.ds(16 * j, 16)]
                        bs = b * sgv[j]
                        cur = acc_m[dl, 0, pl.ds(16 * j, 16)]
                        acc_m[dl, 0, pl.ds(16 * j, 16)] = jnp.maximum(cur, bs)
                        plsc.addupdate(acc_s.at[dl, 0, pl.ds(16 * j, 16)], b)
                    return 0
                lax.fori_loop(0, gl, do_edge, 0)
                return 0

            lax.fori_loop(0, ng, do_group, 0)

        def do_chunk(k, cnt_vec):
            pend = jnp.max(cnt_vec)

            @pl.when(pend > _CAP - _ECH)
            def _():
                flush(pend)
            cnt_vec = jnp.where(cnt_vec > _CAP - _ECH,
                                jnp.zeros((16,), jnp.int32), cnt_vec)

            pltpu.sync_copy(dst_h.at[pl.ds(k * _ECH, _ECH)], dbuf)
            pltpu.sync_copy(src_h.at[pl.ds(k * _ECH, _ECH)], sbuf)

            def scan(i, cv):
                d = dbuf[pl.ds(16 * i, 16)]
                s = sbuf[pl.ds(16 * i, 16)]
                msk = (d >= base) & (d < base + _NT)
                m32 = msk.astype(jnp.int32)
                pos = cv + plsc.cumsum(m32) - 1
                plsc.store_scatter(dcomp, [pos], d - base, mask=msk)
                plsc.store_scatter(icomp, [pos], s * _HROW + toff, mask=msk)
                return cv + plsc.all_reduce_population_count(msk)
            return lax.fori_loop(0, _ECH // 16, scan, cnt_vec)

        cnt_vec = lax.fori_loop(0, _NCH, do_chunk,
                                jnp.zeros((16,), jnp.int32))
        tail = jnp.max(cnt_vec)

        @pl.when(tail > 0)
        def _():
            flush(tail)

        @pl.when(v == 0)
        def _():
            pltpu.sync_copy(acc_m.at[pl.ds(0, _NT)],
                            m1.at[pl.ds(base, _NT), pl.ds(c, 1)])
            pltpu.sync_copy(acc_s.at[pl.ds(0, _NT)],
                            s1.at[pl.ds(base, _NT), pl.ds(c, 1)])

        @pl.when(v == 1)
        def _():
            pltpu.sync_copy(acc_m.at[pl.ds(0, _NT)],
                            m2.at[pl.ds(base, _NT), pl.ds(c, 1)])
            pltpu.sync_copy(acc_s.at[pl.ds(0, _NT)],
                            s2.at[pl.ds(base, _NT), pl.ds(c, 1)])
        return 0

    lax.fori_loop(0, 2 * _NC_F, do_pass, 0)


_out_sd = jax.ShapeDtypeStruct((_NPAD, _NC_F, _FC), jnp.float32)

_sc_call = functools.partial(
    pl.kernel,
    mesh=plsc.VectorSubcoreMesh(core_axis_name="c", subcore_axis_name="s"),
    compiler_params=pltpu.CompilerParams(needs_layout_passes=False),
    out_type=[_out_sd, _out_sd, _out_sd, _out_sd],
    scratch_types=[
        pltpu.VMEM((_ECH,), jnp.int32),
        pltpu.VMEM((_ECH,), jnp.int32),
        pltpu.VMEM((_CAP + 128,), jnp.int32),
        pltpu.VMEM((_CAP + 128,), jnp.int32),
        pltpu.VMEM((_NT, 1, _FC), jnp.float32),
        pltpu.VMEM((_NT, 1, _FC), jnp.float32),
        pltpu.VMEM((_G, _FC), jnp.float32),
        pltpu.VMEM((2, _NC_F, _FC), jnp.float32),
        pltpu.SemaphoreType.DMA,
    ],
)(_sc_body)


def kernel(x, edge_index, batch, W0, b0, g0, be0, W1, b1, g1, be1,
           W2, b2, g2, be2, Wr, br):
    src = edge_index[0]
    dst = edge_index[1]

    Wcat = jnp.concatenate([
        W0,
        W1[:_F] - W1[_F:], W1[_F:],
        W2[:_F] - W2[_F:], W2[_F:],
    ], axis=1)
    H = _matmul(x, Wcat, 400, 512)
    H0 = H[:, 0 * _F:1 * _F]
    A1 = H[:, 1 * _F:2 * _F]
    A2 = H[:, 3 * _F:4 * _F]

    hflat = H.reshape(_N * _HROW, _FC)
    sigs = jnp.stack([jnp.sign(g1).reshape(_NC_F, _FC),
                      jnp.sign(g2).reshape(_NC_F, _FC)])
    m1p, s1p, m2p, s2p = _sc_call(hflat, dst, src, sigs)
    M1 = m1p.reshape(_NPAD, _F)[:_N]
    S1 = s1p.reshape(_NPAD, _F)[:_N]
    M2 = m2p.reshape(_NPAD, _F)[:_N]
    S2 = s2p.reshape(_NPAD, _F)[:_N]

    cnt_dst = jax.ops.segment_sum(jnp.ones((_E,), jnp.float32), dst,
                                  num_segments=_N)
    cnt_src = jax.ops.segment_sum(jnp.ones((_E,), jnp.float32), src,
                                  num_segments=_N)

    P = (batch[None, :] == jnp.arange(_NG)[:, None]).astype(jnp.float32)
    Pn = P / jnp.maximum(P.sum(1, keepdims=True), 1.0)

    mu0 = jnp.mean(H0, axis=0) + b0
    var0 = jnp.mean((H0 + b0[None, :] - mu0[None, :]) ** 2, axis=0)
    s0 = g0 * jax.lax.rsqrt(var0 + _EPS)
    t0 = (b0 - mu0) * s0 + be0
    p0 = (Pn @ H0) * s0[None, :] + t0[None, :]

    def edgeconv2(A, B, S, M, b, g, be):
        sumA = cnt_dst @ A
        sumB = cnt_src @ B
        sumA2 = cnt_dst @ (A * A)
        sumB2 = cnt_src @ (B * B)
        cross = jnp.sum(A * S, axis=0)
        mu_nb = (sumA + sumB) / _E
        var = (sumA2 + sumB2 + 2.0 * cross) / _E - mu_nb ** 2
        s = g * jax.lax.rsqrt(var + _EPS)
        t = (b - mu_nb) * s + be
        mask = (cnt_dst > 0)[:, None]
        Mz = jnp.where(mask, M, 0.0)
        agg = A * s[None, :] + t[None, :] + jnp.abs(s)[None, :] * Mz
        agg = jnp.where(mask & jnp.isfinite(agg), agg, 0.0)
        return Pn @ agg

    B1 = H[:, 2 * _F:3 * _F]
    B2 = H[:, 4 * _F:5 * _F]
    p1 = edgeconv2(A1, B1, S1, M1, b1, g1, be1)
    p2 = edgeconv2(A2, B2, S2, M2, b2, g2, be2)
    acc = p0 + p1 + p2
    return acc @ Wr + br
